# baseline (speedup 1.0000x reference)
import jax
import jax.numpy as jnp
from jax.experimental import pallas as pl
from jax.experimental.pallas import tpu as pltpu

B = 256
C = 2048
P = 128
NUM_CLASSES = 10000
SCALE = 1.0

_BB = 8
_PS = B // _BB

_NT = 1024
_NC = (NUM_CLASSES + _NT - 1) // _NT


_D = 4


def _x_copies(m, x_ref, x_buf, sem):
    half = _BB // 2
    slot = jax.lax.rem(m, _D)
    c0 = pltpu.make_async_copy(
        x_ref.at[pl.ds(m * _BB, half)],
        x_buf.at[slot, pl.ds(0, half)], sem.at[slot, 0])
    c1 = pltpu.make_async_copy(
        x_ref.at[pl.ds(m * _BB + half, half)],
        x_buf.at[slot, pl.ds(half, half)], sem.at[slot, 1])
    return c0, c1


def _fused_kernel(t_ref, x_ref, w_ref, o1_ref, o2_ref, onk_ref, nw_ref,
                  neck_ref, m_acc, s_acc, t_acc, x_buf, sem):
    n = pl.program_id(0)

    @pl.when(n == 0)
    def _prologue():
        for m in range(_D - 1):
            for c in _x_copies(m, x_ref, x_buf, sem):
                c.start()

    @pl.when(n < _PS)
    def _pool():
        @pl.when(n + _D - 1 < _PS)
        def _():
            for c in _x_copies(n + _D - 1, x_ref, x_buf, sem):
                c.start()
        for c in _x_copies(n, x_ref, x_buf, sem):
            c.wait()
        pooled = jnp.mean(x_buf[jax.lax.rem(n, _D)], axis=1)
        neck_ref[pl.ds(n * _BB, _BB), :] = pooled
        onk_ref[...] = pooled

    @pl.when(n >= _PS)
    def _head():
        k = n - _PS
        tile = jax.lax.dot_general(
            w_ref[...], neck_ref[...],
            dimension_numbers=(((1,), (1,)), ((), ())),
            preferred_element_type=jnp.float32)
        o1_ref[...] = tile
        o2_ref[...] = tile

        row0 = k * _NT
        row = jax.lax.broadcasted_iota(jnp.int32, (_NT, B), 0)
        neg = jnp.float32(-jnp.inf)
        masked = jnp.where(row0 + row < NUM_CLASSES, tile, neg)
        cmax = jnp.max(masked, axis=0, keepdims=True)
        sexp = jnp.sum(jnp.exp(masked - cmax), axis=0, keepdims=True)
        hit = row == (t_ref[...] - row0)
        ltv = jnp.max(jnp.where(hit, tile, neg), axis=0, keepdims=True)

        @pl.when(k == 0)
        def _():
            m_acc[...] = cmax
            s_acc[...] = sexp
            t_acc[...] = ltv

        @pl.when(k > 0)
        def _():
            m_old = m_acc[...]
            m_new = jnp.maximum(m_old, cmax)
            s_acc[...] = (s_acc[...] * jnp.exp(m_old - m_new)
                          + sexp * jnp.exp(cmax - m_new))
            m_acc[...] = m_new
            t_acc[...] = jnp.maximum(t_acc[...], ltv)

        @pl.when(k == _NC - 1)
        def _():
            right_prob = jnp.exp(t_acc[...] - m_acc[...]) / s_acc[...]
            var_sl = jnp.zeros_like(right_prob)
            con = right_prob / (var_sl * 1e4)
            ri = jnp.tanh(1.2 * con)
            nw_ref[...] = (jnp.float32(B) * ri) / jnp.sum(ri, axis=1,
                                                          keepdims=True)


def kernel(features, targets, weight):
    x = features.transpose(0, 2, 3, 1).reshape(B, P, C)
    t2 = targets.astype(jnp.int32).reshape(1, B)
    o1, o2, neck, nw = pl.pallas_call(
        _fused_kernel,
        grid=(_PS + _NC,),
        in_specs=[
            pl.BlockSpec((1, B), lambda n: (0, 0)),
            pl.BlockSpec(memory_space=pl.ANY),
            pl.BlockSpec((_NT, C),
                         lambda n: (jnp.maximum(n - _PS, 0), 0)),
        ],
        out_specs=[
            pl.BlockSpec((_NT, B), lambda n: (jnp.maximum(n - _PS, 0), 0)),
            pl.BlockSpec((_NT, B), lambda n: (jnp.maximum(n - _PS, 0), 0)),
            pl.BlockSpec((_BB, C), lambda n: (jnp.minimum(n, _PS - 1), 0)),
            pl.BlockSpec((1, B), lambda n: (0, 0)),
        ],
        out_shape=[
            jax.ShapeDtypeStruct((NUM_CLASSES, B), jnp.float32),
            jax.ShapeDtypeStruct((NUM_CLASSES, B), jnp.float32),
            jax.ShapeDtypeStruct((B, C), jnp.float32),
            jax.ShapeDtypeStruct((1, B), jnp.float32),
        ],
        scratch_shapes=[
            pltpu.VMEM((B, C), jnp.float32),
            pltpu.VMEM((1, B), jnp.float32),
            pltpu.VMEM((1, B), jnp.float32),
            pltpu.VMEM((1, B), jnp.float32),
            pltpu.VMEM((_D, _BB, P, C), jnp.float32),
            pltpu.SemaphoreType.DMA((_D, 2)),
        ],
        compiler_params=pltpu.CompilerParams(
            dimension_semantics=("arbitrary",),
            vmem_limit_bytes=56 * 1024 * 1024),
        name="pool_linear_softmax",
    )(t2, x, weight)

    return o1.T, o2.T, neck, nw


# speedup vs baseline: 1.0234x; 1.0234x over previous
import jax
import jax.numpy as jnp
from jax.experimental import pallas as pl
from jax.experimental.pallas import tpu as pltpu

B = 256
C = 2048
P = 128
NUM_CLASSES = 10000
SCALE = 1.0

_BB = 8
_PS = B // _BB

_NT = 1024
_NC = (NUM_CLASSES + _NT - 1) // _NT


_D = 4


def _x_copies(m, x_ref, x_buf, sem):
    slot = jax.lax.rem(m, _D)
    c0 = pltpu.make_async_copy(
        x_ref.at[pl.ds(m * _BB, _BB)], x_buf.at[slot], sem.at[slot, 0])
    return (c0,)


def _fused_kernel(t_ref, x_ref, w_ref, o1_ref, o2_ref, onk_ref, nw_ref,
                  neck_ref, m_acc, s_acc, t_acc, x_buf, sem):
    n = pl.program_id(0)

    @pl.when(n == 0)
    def _prologue():
        for m in range(_D - 1):
            for c in _x_copies(m, x_ref, x_buf, sem):
                c.start()

    @pl.when(n < _PS)
    def _pool():
        @pl.when(n + _D - 1 < _PS)
        def _():
            for c in _x_copies(n + _D - 1, x_ref, x_buf, sem):
                c.start()
        for c in _x_copies(n, x_ref, x_buf, sem):
            c.wait()
        pooled = jnp.mean(x_buf[jax.lax.rem(n, _D)], axis=1)
        neck_ref[pl.ds(n * _BB, _BB), :] = pooled
        onk_ref[...] = pooled

    @pl.when(n >= _PS)
    def _head():
        k = n - _PS
        tile = jax.lax.dot_general(
            w_ref[...], neck_ref[...],
            dimension_numbers=(((1,), (1,)), ((), ())),
            preferred_element_type=jnp.float32)
        o1_ref[...] = tile
        o2_ref[...] = tile

        row0 = k * _NT
        row = jax.lax.broadcasted_iota(jnp.int32, (_NT, B), 0)
        neg = jnp.float32(-jnp.inf)
        masked = jnp.where(row0 + row < NUM_CLASSES, tile, neg)
        cmax = jnp.max(masked, axis=0, keepdims=True)
        sexp = jnp.sum(jnp.exp(masked - cmax), axis=0, keepdims=True)
        hit = row == (t_ref[...] - row0)
        ltv = jnp.max(jnp.where(hit, tile, neg), axis=0, keepdims=True)

        @pl.when(k == 0)
        def _():
            m_acc[...] = cmax
            s_acc[...] = sexp
            t_acc[...] = ltv

        @pl.when(k > 0)
        def _():
            m_old = m_acc[...]
            m_new = jnp.maximum(m_old, cmax)
            s_acc[...] = (s_acc[...] * jnp.exp(m_old - m_new)
                          + sexp * jnp.exp(cmax - m_new))
            m_acc[...] = m_new
            t_acc[...] = jnp.maximum(t_acc[...], ltv)

        @pl.when(k == _NC - 1)
        def _():
            right_prob = jnp.exp(t_acc[...] - m_acc[...]) / s_acc[...]
            var_sl = jnp.zeros_like(right_prob)
            con = right_prob / (var_sl * 1e4)
            ri = jnp.tanh(1.2 * con)
            nw_ref[...] = (jnp.float32(B) * ri) / jnp.sum(ri, axis=1,
                                                          keepdims=True)


def kernel(features, targets, weight):
    x = features.transpose(0, 2, 3, 1).reshape(B, P, C)
    t2 = targets.astype(jnp.int32).reshape(1, B)
    o1, o2, neck, nw = pl.pallas_call(
        _fused_kernel,
        grid=(_PS + _NC,),
        in_specs=[
            pl.BlockSpec((1, B), lambda n: (0, 0)),
            pl.BlockSpec(memory_space=pl.ANY),
            pl.BlockSpec((_NT, C),
                         lambda n: (jnp.maximum(n - _PS, 0), 0)),
        ],
        out_specs=[
            pl.BlockSpec((_NT, B), lambda n: (jnp.maximum(n - _PS, 0), 0)),
            pl.BlockSpec((_NT, B), lambda n: (jnp.maximum(n - _PS, 0), 0)),
            pl.BlockSpec((_BB, C), lambda n: (jnp.minimum(n, _PS - 1), 0)),
            pl.BlockSpec((1, B), lambda n: (0, 0)),
        ],
        out_shape=[
            jax.ShapeDtypeStruct((NUM_CLASSES, B), jnp.float32),
            jax.ShapeDtypeStruct((NUM_CLASSES, B), jnp.float32),
            jax.ShapeDtypeStruct((B, C), jnp.float32),
            jax.ShapeDtypeStruct((1, B), jnp.float32),
        ],
        scratch_shapes=[
            pltpu.VMEM((B, C), jnp.float32),
            pltpu.VMEM((1, B), jnp.float32),
            pltpu.VMEM((1, B), jnp.float32),
            pltpu.VMEM((1, B), jnp.float32),
            pltpu.VMEM((_D, _BB, P, C), jnp.float32),
            pltpu.SemaphoreType.DMA((_D, 2)),
        ],
        compiler_params=pltpu.CompilerParams(
            dimension_semantics=("arbitrary",),
            vmem_limit_bytes=56 * 1024 * 1024),
        name="pool_linear_softmax",
    )(t2, x, weight)

    return o1.T, o2.T, neck, nw


# speedup vs baseline: 1.0349x; 1.0112x over previous
import jax
import jax.numpy as jnp
from jax.experimental import pallas as pl
from jax.experimental.pallas import tpu as pltpu

B = 256
C = 2048
P = 128
NUM_CLASSES = 10000
SCALE = 1.0

_BB = 8
_PS = B // _BB

_NT = 1024
_NC = (NUM_CLASSES + _NT - 1) // _NT


def _fused_kernel(t_ref, x_ref, w_ref, o1_ref, o2_ref, onk_ref, nw_ref,
                  neck_ref, m_acc, s_acc, t_acc):
    n = pl.program_id(0)

    @pl.when(n < _PS)
    def _pool():
        pooled = jnp.mean(x_ref[...], axis=1)
        neck_ref[pl.ds(n * _BB, _BB), :] = pooled
        onk_ref[...] = pooled

    @pl.when(n >= _PS)
    def _head():
        k = n - _PS
        tile = jax.lax.dot_general(
            w_ref[...], neck_ref[...],
            dimension_numbers=(((1,), (1,)), ((), ())),
            preferred_element_type=jnp.float32)
        o1_ref[...] = tile
        o2_ref[...] = tile

        row0 = k * _NT
        row = jax.lax.broadcasted_iota(jnp.int32, (_NT, B), 0)
        neg = jnp.float32(-jnp.inf)
        masked = jnp.where(row0 + row < NUM_CLASSES, tile, neg)
        cmax = jnp.max(masked, axis=0, keepdims=True)
        sexp = jnp.sum(jnp.exp(masked - cmax), axis=0, keepdims=True)
        hit = row == (t_ref[...] - row0)
        ltv = jnp.max(jnp.where(hit, tile, neg), axis=0, keepdims=True)

        @pl.when(k == 0)
        def _():
            m_acc[...] = cmax
            s_acc[...] = sexp
            t_acc[...] = ltv

        @pl.when(k > 0)
        def _():
            m_old = m_acc[...]
            m_new = jnp.maximum(m_old, cmax)
            s_acc[...] = (s_acc[...] * jnp.exp(m_old - m_new)
                          + sexp * jnp.exp(cmax - m_new))
            m_acc[...] = m_new
            t_acc[...] = jnp.maximum(t_acc[...], ltv)

        @pl.when(k == _NC - 1)
        def _():
            right_prob = jnp.exp(t_acc[...] - m_acc[...]) / s_acc[...]
            var_sl = jnp.zeros_like(right_prob)
            con = right_prob / (var_sl * 1e4)
            ri = jnp.tanh(1.2 * con)
            nw_ref[...] = (jnp.float32(B) * ri) / jnp.sum(ri, axis=1,
                                                          keepdims=True)


def kernel(features, targets, weight):
    x = features.transpose(0, 2, 3, 1).reshape(B, P, C)
    t2 = targets.astype(jnp.int32).reshape(1, B)
    o1, o2, neck, nw = pl.pallas_call(
        _fused_kernel,
        grid=(_PS + _NC,),
        in_specs=[
            pl.BlockSpec((1, B), lambda n: (0, 0)),
            pl.BlockSpec((_BB, P, C),
                         lambda n: (jnp.minimum(n, _PS - 1), 0, 0)),
            pl.BlockSpec((_NT, C),
                         lambda n: (jnp.maximum(n - _PS, 0), 0)),
        ],
        out_specs=[
            pl.BlockSpec((_NT, B), lambda n: (jnp.maximum(n - _PS, 0), 0)),
            pl.BlockSpec((_NT, B), lambda n: (jnp.maximum(n - _PS, 0), 0)),
            pl.BlockSpec((_BB, C), lambda n: (jnp.minimum(n, _PS - 1), 0)),
            pl.BlockSpec((1, B), lambda n: (0, 0)),
        ],
        out_shape=[
            jax.ShapeDtypeStruct((NUM_CLASSES, B), jnp.float32),
            jax.ShapeDtypeStruct((NUM_CLASSES, B), jnp.float32),
            jax.ShapeDtypeStruct((B, C), jnp.float32),
            jax.ShapeDtypeStruct((1, B), jnp.float32),
        ],
        scratch_shapes=[
            pltpu.VMEM((B, C), jnp.float32),
            pltpu.VMEM((1, B), jnp.float32),
            pltpu.VMEM((1, B), jnp.float32),
            pltpu.VMEM((1, B), jnp.float32),
        ],
        compiler_params=pltpu.CompilerParams(
            dimension_semantics=("arbitrary",),
            vmem_limit_bytes=56 * 1024 * 1024),
        name="pool_linear_softmax",
    )(t2, x, weight)

    return o1.T, o2.T, neck, nw
